# Initial kernel scaffold; baseline (speedup 1.0000x reference)
import jax
import jax.numpy as jnp
from jax.experimental import pallas as pl


def kernel(x_nchw, skip_nchw, weight, bias):
    raise NotImplementedError("write your pallas kernel here")


# speedup vs baseline: 1.5393x; 1.5393x over previous
import jax
import jax.numpy as jnp
from jax.experimental import pallas as pl
from jax.experimental.pallas import tpu as pltpu


def _fpn_kernel_body(x_ref, s_ref, w_ref, u_ref, b_ref, o_ref, *, thl):
    w = w_ref[...]
    u = u_ref[...]
    b = b_ref[...]
    for hl in range(thl):
        xr = x_ref[0, :, hl, :]
        xu = jnp.dot(xr, u, preferred_element_type=jnp.float32) + b
        for j in range(2):
            s = s_ref[0, :, 2 * hl + j, :]
            conv = jnp.dot(w, s, preferred_element_type=jnp.float32)
            o_ref[0, :, 2 * hl + j, :] = (conv + xu).astype(o_ref.dtype)


def kernel(x_nchw, skip_nchw, weight, bias):
    N, Cp, H, W = x_nchw.shape
    _, Cs, Hs, Ws = skip_nchw.shape

    TH = 16
    THL = TH // 2
    grid = (N, Hs // TH)

    w2 = weight.reshape(Cp, Cs)
    u = jnp.repeat(jnp.eye(W, dtype=x_nchw.dtype), 2, axis=1)
    b2 = jnp.broadcast_to(bias.astype(jnp.float32)[:, None], (Cp, Ws))

    import functools
    body = functools.partial(_fpn_kernel_body, thl=THL)

    out = pl.pallas_call(
        body,
        out_shape=jax.ShapeDtypeStruct((N, Cp, Hs, Ws), x_nchw.dtype),
        grid=grid,
        in_specs=[
            pl.BlockSpec((1, Cp, THL, W), lambda n, t: (n, 0, t, 0)),
            pl.BlockSpec((1, Cs, TH, Ws), lambda n, t: (n, 0, t, 0)),
            pl.BlockSpec((Cp, Cs), lambda n, t: (0, 0)),
            pl.BlockSpec((W, Ws), lambda n, t: (0, 0)),
            pl.BlockSpec((Cp, Ws), lambda n, t: (0, 0)),
        ],
        out_specs=pl.BlockSpec((1, Cp, TH, Ws), lambda n, t: (n, 0, t, 0)),
        compiler_params=pltpu.CompilerParams(
            dimension_semantics=("parallel", "parallel"),
            vmem_limit_bytes=64 * 2**20,
        ),
    )(x_nchw, skip_nchw, w2, u, b2)
    return out


# speedup vs baseline: 1.9081x; 1.2396x over previous
import jax
import jax.numpy as jnp
from jax.experimental import pallas as pl
from jax.experimental.pallas import tpu as pltpu


def _fpn_kernel_body(x_ref, s_ref, w_ref, u_ref, b_ref, o_ref, *, thl):
    w = w_ref[...]
    u = u_ref[...]
    b = b_ref[...]
    s_t = jnp.swapaxes(s_ref[0].astype(jnp.bfloat16), 0, 1)
    x_t = jnp.swapaxes(x_ref[0].astype(jnp.bfloat16), 0, 1)
    for hl in range(thl):
        xu = jnp.dot(x_t[hl], u, preferred_element_type=jnp.float32) + b
        for j in range(2):
            conv = jnp.dot(w, s_t[2 * hl + j],
                           preferred_element_type=jnp.float32)
            o_ref[0, :, 2 * hl + j, :] = (conv + xu).astype(o_ref.dtype)


def kernel(x_nchw, skip_nchw, weight, bias):
    N, Cp, H, W = x_nchw.shape
    _, Cs, Hs, Ws = skip_nchw.shape

    TH = 16
    THL = TH // 2
    grid = (N, Hs // TH)

    w2 = weight.reshape(Cp, Cs).astype(jnp.bfloat16)
    u = jnp.repeat(jnp.eye(W, dtype=jnp.bfloat16), 2, axis=1)
    b2 = jnp.broadcast_to(bias.astype(jnp.float32)[:, None], (Cp, Ws))

    import functools
    body = functools.partial(_fpn_kernel_body, thl=THL)

    out = pl.pallas_call(
        body,
        out_shape=jax.ShapeDtypeStruct((N, Cp, Hs, Ws), x_nchw.dtype),
        grid=grid,
        in_specs=[
            pl.BlockSpec((1, Cp, THL, W), lambda n, t: (n, 0, t, 0)),
            pl.BlockSpec((1, Cs, TH, Ws), lambda n, t: (n, 0, t, 0)),
            pl.BlockSpec((Cp, Cs), lambda n, t: (0, 0)),
            pl.BlockSpec((W, Ws), lambda n, t: (0, 0)),
            pl.BlockSpec((Cp, Ws), lambda n, t: (0, 0)),
        ],
        out_specs=pl.BlockSpec((1, Cp, TH, Ws), lambda n, t: (n, 0, t, 0)),
        compiler_params=pltpu.CompilerParams(
            dimension_semantics=("parallel", "parallel"),
            vmem_limit_bytes=64 * 2**20,
        ),
    )(x_nchw, skip_nchw, w2, u, b2)
    return out


# speedup vs baseline: 2.0650x; 1.0823x over previous
import functools

import jax
import jax.numpy as jnp
from jax.experimental import pallas as pl
from jax.experimental.pallas import tpu as pltpu


def _fpn_kernel_body(x_ref, s_ref, w_ref, u_ref, b_ref, o_ref, *, thl2):
    w = w_ref[...]
    u = u_ref[...]
    b = b_ref[...]
    ws = o_ref.shape[-1]
    s_t = jnp.swapaxes(s_ref[0].astype(jnp.bfloat16), 0, 1)
    x_t = jnp.swapaxes(x_ref[0].astype(jnp.bfloat16), 0, 1)
    for i in range(thl2):
        xu2 = jnp.dot(x_t[i], u, preferred_element_type=jnp.float32) + b
        for q in range(4):
            h = 4 * i + q
            xu = jax.lax.slice_in_dim(xu2, (q // 2) * ws, (q // 2 + 1) * ws,
                                      axis=1)
            conv = jnp.dot(w, s_t[h], preferred_element_type=jnp.float32)
            o_ref[0, :, h, :] = (conv + xu).astype(o_ref.dtype)


def kernel(x_nchw, skip_nchw, weight, bias):
    N, Cp, H, W = x_nchw.shape
    _, Cs, Hs, Ws = skip_nchw.shape

    TH = 32
    THL2 = TH // 4
    grid = (N, Hs // TH)

    x_pairs = x_nchw.reshape(N, Cp, H // 2, 2 * W)

    w2 = weight.reshape(Cp, Cs).astype(jnp.bfloat16)
    u1 = jnp.repeat(jnp.eye(W, dtype=jnp.bfloat16), 2, axis=1)
    u2 = jnp.kron(jnp.eye(2, dtype=jnp.bfloat16), u1)
    b2 = jnp.broadcast_to(bias.astype(jnp.float32)[:, None], (Cp, 4 * W))

    body = functools.partial(_fpn_kernel_body, thl2=THL2)

    out = pl.pallas_call(
        body,
        out_shape=jax.ShapeDtypeStruct((N, Cp, Hs, Ws), x_nchw.dtype),
        grid=grid,
        in_specs=[
            pl.BlockSpec((1, Cp, THL2, 2 * W), lambda n, t: (n, 0, t, 0)),
            pl.BlockSpec((1, Cs, TH, Ws), lambda n, t: (n, 0, t, 0)),
            pl.BlockSpec((Cp, Cs), lambda n, t: (0, 0)),
            pl.BlockSpec((2 * W, 4 * W), lambda n, t: (0, 0)),
            pl.BlockSpec((Cp, 4 * W), lambda n, t: (0, 0)),
        ],
        out_specs=pl.BlockSpec((1, Cp, TH, Ws), lambda n, t: (n, 0, t, 0)),
        compiler_params=pltpu.CompilerParams(
            dimension_semantics=("parallel", "parallel"),
            vmem_limit_bytes=64 * 2**20,
        ),
    )(x_pairs, skip_nchw, w2, u2, b2)
    return out


# speedup vs baseline: 2.4062x; 1.1652x over previous
import functools

import jax
import jax.numpy as jnp
from jax.experimental import pallas as pl
from jax.experimental.pallas import tpu as pltpu


def _fpn_kernel_body(x_ref, s_ref, w_ref, u_ref, b_ref, o_ref, *, thl):
    w = w_ref[...]
    u = u_ref[...]
    b = b_ref[...]
    s_t = jnp.swapaxes(s_ref[0].astype(jnp.bfloat16), 0, 1)
    x3 = x_ref[0].astype(jnp.bfloat16)
    for hl in range(thl):
        xu = jax.lax.dot_general(
            x3[hl], u, (((0,), (0,)), ((), ())),
            preferred_element_type=jnp.float32)
        xu = xu + b
        for j in range(2):
            conv = jnp.dot(w, s_t[2 * hl + j],
                           preferred_element_type=jnp.float32)
            o_ref[0, :, 2 * hl + j, :] = (conv + xu).astype(o_ref.dtype)


def kernel(x_nchw, skip_nchw, weight, bias):
    N, Cp, H, W = x_nchw.shape
    _, Cs, Hs, Ws = skip_nchw.shape

    TH = 32
    THL = TH // 2
    grid = (N, Hs // TH)

    x_nhwc = jnp.transpose(x_nchw, (0, 2, 3, 1))

    w2 = weight.reshape(Cp, Cs).astype(jnp.bfloat16)
    u = jnp.repeat(jnp.eye(W, dtype=jnp.bfloat16), 2, axis=1)
    b2 = jnp.broadcast_to(bias.astype(jnp.float32)[:, None], (Cp, 2 * W))

    body = functools.partial(_fpn_kernel_body, thl=THL)

    out = pl.pallas_call(
        body,
        out_shape=jax.ShapeDtypeStruct((N, Cp, Hs, Ws), x_nchw.dtype),
        grid=grid,
        in_specs=[
            pl.BlockSpec((1, THL, W, Cp), lambda n, t: (n, t, 0, 0)),
            pl.BlockSpec((1, Cs, TH, Ws), lambda n, t: (n, 0, t, 0)),
            pl.BlockSpec((Cp, Cs), lambda n, t: (0, 0)),
            pl.BlockSpec((W, 2 * W), lambda n, t: (0, 0)),
            pl.BlockSpec((Cp, 2 * W), lambda n, t: (0, 0)),
        ],
        out_specs=pl.BlockSpec((1, Cp, TH, Ws), lambda n, t: (n, 0, t, 0)),
        compiler_params=pltpu.CompilerParams(
            dimension_semantics=("parallel", "parallel"),
            vmem_limit_bytes=64 * 2**20,
        ),
    )(x_nhwc, skip_nchw, w2, u, b2)
    return out
